# baseline (speedup 1.0000x reference)
import functools

import jax
import jax.numpy as jnp
from jax import lax
from jax.experimental import pallas as pl
from jax.experimental.pallas import tpu as pltpu
from jax.experimental.pallas import tpu_sc as plsc

_EMBED = 32
_NC = 2
_NS = 16
_NW = _NC * _NS
_NBUF = 8


@functools.lru_cache(maxsize=None)
def _build(batch: int, hist: int):
    r_per_w = batch // _NW
    n_chunks = r_per_w
    assert batch % _NW == 0
    assert n_chunks % _NBUF == 0 and n_chunks // _NBUF >= 2

    mesh = plsc.VectorSubcoreMesh(core_axis_name="c", subcore_axis_name="s")

    scratch = (
        [pltpu.VMEM((hist,), jnp.int32) for _ in range(_NBUF)]
        + [pltpu.VMEM((hist, _EMBED), jnp.float32) for _ in range(_NBUF)]
        + [pltpu.SemaphoreType.DMA for _ in range(2 * _NBUF)]
    )

    @functools.partial(
        pl.kernel,
        mesh=mesh,
        out_type=jax.ShapeDtypeStruct((batch, hist, _EMBED), jnp.float32),
        compiler_params=pltpu.CompilerParams(use_tc_tiling_on_sc=False),
        scratch_types=scratch,
    )
    def gather_k(idx_hbm, table_hbm, out_hbm, *refs):
        idx_v = refs[0:_NBUF]
        rows_v = refs[_NBUF:2 * _NBUF]
        gsem = refs[2 * _NBUF:3 * _NBUF]
        ssem = refs[3 * _NBUF:4 * _NBUF]

        wid = lax.axis_index("s") * _NC + lax.axis_index("c")
        base = wid * r_per_w

        def fire(g, s, wait_store):
            if wait_store:
                pltpu.make_async_copy(
                    rows_v[s], out_hbm.at[base + g - _NBUF],
                    ssem[s]).wait()
            pltpu.sync_copy(idx_hbm.at[base + g], idx_v[s])
            pltpu.async_copy(table_hbm.at[idx_v[s]], rows_v[s], gsem[s])

        def drain(g, s):
            pltpu.make_async_copy(
                table_hbm.at[idx_v[s]], rows_v[s], gsem[s]).wait()
            pltpu.async_copy(rows_v[s], out_hbm.at[base + g], ssem[s])

        for s in range(_NBUF):
            fire(s, s, wait_store=False)

        def body(j, carry):
            g0 = j * _NBUF
            for s in range(_NBUF):
                drain(g0 + s, s)
            for s in range(_NBUF):
                fire(g0 + _NBUF + s, s, wait_store=True)
            return carry

        lax.fori_loop(0, n_chunks // _NBUF - 1, body, 0)

        g0 = n_chunks - _NBUF
        for s in range(_NBUF):
            drain(g0 + s, s)
        for s in range(_NBUF):
            pltpu.make_async_copy(
                rows_v[s], out_hbm.at[base + g0 + s], ssem[s]).wait()

    return gather_k


def kernel(a, table):
    batch, hist = a.shape
    return _build(batch, hist)(a.astype(jnp.int32), table)


# speedup vs baseline: 1.0607x; 1.0607x over previous
import functools

import jax
import jax.numpy as jnp
from jax import lax
from jax.experimental import pallas as pl
from jax.experimental.pallas import tpu as pltpu
from jax.experimental.pallas import tpu_sc as plsc

_EMBED = 32
_NC = 2
_NS = 16
_NW = _NC * _NS
_CHUNK = 800
_NBUF = 4


@functools.lru_cache(maxsize=None)
def _build(B: int):
    b_per_w = B // _NW
    n_chunks = b_per_w // _CHUNK
    assert B % (8 * _NW) == 0 and b_per_w % _CHUNK == 0
    assert n_chunks % _NBUF == 0 and n_chunks // _NBUF >= 2

    mesh = plsc.VectorSubcoreMesh(core_axis_name="c", subcore_axis_name="s")

    scratch = (
        [pltpu.VMEM((_CHUNK,), jnp.int32) for _ in range(_NBUF)]
        + [pltpu.VMEM((_CHUNK, _EMBED), jnp.float32) for _ in range(_NBUF)]
        + [pltpu.SemaphoreType.DMA for _ in range(2 * _NBUF)]
    )

    @functools.partial(
        pl.kernel,
        mesh=mesh,
        out_type=jax.ShapeDtypeStruct((B, _EMBED), jnp.float32),
        compiler_params=pltpu.CompilerParams(use_tc_tiling_on_sc=False),
        scratch_types=scratch,
    )
    def gather_k(idx_hbm, table_hbm, out_hbm, *refs):
        idx_v = refs[0:_NBUF]
        rows_v = refs[_NBUF:2 * _NBUF]
        gsem = refs[2 * _NBUF:3 * _NBUF]
        ssem = refs[3 * _NBUF:4 * _NBUF]

        wid = lax.axis_index("s") * _NC + lax.axis_index("c")
        base = wid * b_per_w

        def out_slice(g):
            off = base + g * _CHUNK
            return out_hbm.at[pl.ds(off, _CHUNK)]

        def fire(g, s, wait_store):
            if wait_store:
                pltpu.make_async_copy(
                    rows_v[s], out_slice(g - _NBUF), ssem[s]).wait()
            off = base + g * _CHUNK
            pltpu.sync_copy(idx_hbm.at[pl.ds(off, _CHUNK)], idx_v[s])
            pltpu.async_copy(table_hbm.at[idx_v[s]], rows_v[s], gsem[s])

        def drain(g, s):
            pltpu.make_async_copy(
                table_hbm.at[idx_v[s]], rows_v[s], gsem[s]).wait()
            pltpu.async_copy(rows_v[s], out_slice(g), ssem[s])

        for s in range(_NBUF):
            fire(s, s, wait_store=False)

        def body(j, carry):
            g0 = j * _NBUF
            for s in range(_NBUF):
                drain(g0 + s, s)
            for s in range(_NBUF):
                fire(g0 + _NBUF + s, s, wait_store=True)
            return carry

        lax.fori_loop(0, n_chunks // _NBUF - 1, body, 0)

        g0 = n_chunks - _NBUF
        for s in range(_NBUF):
            drain(g0 + s, s)
        for s in range(_NBUF):
            pltpu.make_async_copy(
                rows_v[s], out_slice(g0 + s), ssem[s]).wait()

    return gather_k


def kernel(a, table):
    batch, hist = a.shape
    half = batch // 2
    Bh = half * hist
    gather = _build(Bh)
    outs = []
    for part in (a[:half], a[half:]):
        idx = part.reshape(Bh).astype(jnp.int32)
        outs.append(gather(idx, table).reshape(half, hist, _EMBED))
    return jnp.concatenate(outs, axis=0)


# speedup vs baseline: 1.1227x; 1.0585x over previous
import functools

import jax
import jax.numpy as jnp
from jax import lax
from jax.experimental import pallas as pl
from jax.experimental.pallas import tpu as pltpu
from jax.experimental.pallas import tpu_sc as plsc

_EMBED = 32
_NC = 2
_NS = 16
_NW = _NC * _NS
_CHUNK = 800
_NBUF = 4


@functools.lru_cache(maxsize=None)
def _build(B: int):
    b_per_w = B // _NW
    n_chunks = b_per_w // _CHUNK
    assert B % (8 * _NW) == 0 and b_per_w % _CHUNK == 0
    assert n_chunks % _NBUF == 0 and n_chunks // _NBUF >= 2

    mesh = plsc.VectorSubcoreMesh(core_axis_name="c", subcore_axis_name="s")

    scratch = (
        [pltpu.VMEM((_CHUNK,), jnp.int32) for _ in range(_NBUF)]
        + [pltpu.VMEM((_CHUNK, _EMBED), jnp.float32) for _ in range(_NBUF)]
        + [pltpu.SemaphoreType.DMA for _ in range(2 * _NBUF)]
    )

    @functools.partial(
        pl.kernel,
        mesh=mesh,
        out_type=jax.ShapeDtypeStruct((B, _EMBED), jnp.float32),
        compiler_params=pltpu.CompilerParams(use_tc_tiling_on_sc=False),
        scratch_types=scratch,
    )
    def gather_k(idx_hbm, table_hbm, out_hbm, *refs):
        idx_v = refs[0:_NBUF]
        rows_v = refs[_NBUF:2 * _NBUF]
        gsem = refs[2 * _NBUF:3 * _NBUF]
        ssem = refs[3 * _NBUF:4 * _NBUF]

        wid = lax.axis_index("s") * _NC + lax.axis_index("c")
        base = wid * b_per_w

        def out_slice(g):
            off = base + g * _CHUNK
            return out_hbm.at[pl.ds(off, _CHUNK)]

        def fire(g, s, wait_store):
            if wait_store:
                pltpu.make_async_copy(
                    rows_v[s], out_slice(g - _NBUF), ssem[s]).wait()
            off = base + g * _CHUNK
            pltpu.sync_copy(idx_hbm.at[pl.ds(off, _CHUNK)], idx_v[s])
            pltpu.async_copy(table_hbm.at[idx_v[s]], rows_v[s], gsem[s])

        def drain(g, s):
            pltpu.make_async_copy(
                table_hbm.at[idx_v[s]], rows_v[s], gsem[s]).wait()
            pltpu.async_copy(rows_v[s], out_slice(g), ssem[s])

        for s in range(_NBUF):
            fire(s, s, wait_store=False)

        def body(j, carry):
            g0 = j * _NBUF
            for s in range(_NBUF):
                drain(g0 + s, s)
            for s in range(_NBUF):
                fire(g0 + _NBUF + s, s, wait_store=True)
            return carry

        lax.fori_loop(0, n_chunks // _NBUF - 1, body, 0)

        g0 = n_chunks - _NBUF
        for s in range(_NBUF):
            drain(g0 + s, s)
        for s in range(_NBUF):
            pltpu.make_async_copy(
                rows_v[s], out_slice(g0 + s), ssem[s]).wait()

    return gather_k


def kernel(a, table):
    batch, hist = a.shape
    nsplit = 4
    part_b = batch // nsplit
    Bp = part_b * hist
    gather = _build(Bp)
    outs = []
    for i in range(nsplit):
        idx = a[i * part_b:(i + 1) * part_b].reshape(Bp).astype(jnp.int32)
        outs.append(gather(idx, table).reshape(part_b, hist, _EMBED))
    return jnp.concatenate(outs, axis=0)


# speedup vs baseline: 1.1704x; 1.0425x over previous
import functools

import jax
import jax.numpy as jnp
from jax import lax
from jax.experimental import pallas as pl
from jax.experimental.pallas import tpu as pltpu
from jax.experimental.pallas import tpu_sc as plsc

_EMBED = 32
_NC = 2
_NS = 16
_NW = _NC * _NS
_CHUNK = 800
_NBUF = 4


@functools.lru_cache(maxsize=None)
def _build(B: int):
    b_per_w = B // _NW
    n_chunks = b_per_w // _CHUNK
    assert B % (8 * _NW) == 0 and b_per_w % _CHUNK == 0
    assert n_chunks % _NBUF == 0 and n_chunks // _NBUF >= 2

    mesh = plsc.VectorSubcoreMesh(core_axis_name="c", subcore_axis_name="s")

    scratch = (
        [pltpu.VMEM((_CHUNK,), jnp.int32) for _ in range(_NBUF)]
        + [pltpu.VMEM((_CHUNK, _EMBED), jnp.float32) for _ in range(_NBUF)]
        + [pltpu.SemaphoreType.DMA for _ in range(2 * _NBUF)]
    )

    @functools.partial(
        pl.kernel,
        mesh=mesh,
        out_type=jax.ShapeDtypeStruct((B, _EMBED), jnp.float32),
        compiler_params=pltpu.CompilerParams(use_tc_tiling_on_sc=False),
        scratch_types=scratch,
    )
    def gather_k(idx_hbm, table_hbm, out_hbm, *refs):
        idx_v = refs[0:_NBUF]
        rows_v = refs[_NBUF:2 * _NBUF]
        gsem = refs[2 * _NBUF:3 * _NBUF]
        ssem = refs[3 * _NBUF:4 * _NBUF]

        wid = lax.axis_index("s") * _NC + lax.axis_index("c")
        base = wid * b_per_w

        def out_slice(g):
            off = base + g * _CHUNK
            return out_hbm.at[pl.ds(off, _CHUNK)]

        def fire(g, s, wait_store):
            if wait_store:
                pltpu.make_async_copy(
                    rows_v[s], out_slice(g - _NBUF), ssem[s]).wait()
            off = base + g * _CHUNK
            pltpu.sync_copy(idx_hbm.at[pl.ds(off, _CHUNK)], idx_v[s])
            pltpu.async_copy(table_hbm.at[idx_v[s]], rows_v[s], gsem[s])

        def drain(g, s):
            pltpu.make_async_copy(
                table_hbm.at[idx_v[s]], rows_v[s], gsem[s]).wait()
            pltpu.async_copy(rows_v[s], out_slice(g), ssem[s])

        for s in range(_NBUF):
            fire(s, s, wait_store=False)

        def body(j, carry):
            g0 = j * _NBUF
            for s in range(_NBUF):
                drain(g0 + s, s)
            for s in range(_NBUF):
                fire(g0 + _NBUF + s, s, wait_store=True)
            return carry

        lax.fori_loop(0, n_chunks // _NBUF - 1, body, 0)

        g0 = n_chunks - _NBUF
        for s in range(_NBUF):
            drain(g0 + s, s)
        for s in range(_NBUF):
            pltpu.make_async_copy(
                rows_v[s], out_slice(g0 + s), ssem[s]).wait()

    return gather_k


def kernel(a, table):
    batch, hist = a.shape
    nsplit = 8
    part_b = batch // nsplit
    Bp = part_b * hist
    gather = _build(Bp)
    outs = []
    for i in range(nsplit):
        idx = a[i * part_b:(i + 1) * part_b].reshape(Bp).astype(jnp.int32)
        outs.append(gather(idx, table).reshape(part_b, hist, _EMBED))
    return jnp.concatenate(outs, axis=0)
